# baseline (speedup 1.0000x reference)
import functools

import jax
import jax.numpy as jnp
from jax import lax
from jax.experimental import pallas as pl
from jax.experimental.pallas import tpu as pltpu
from jax.experimental.pallas import tpu_sc as plsc

_NC = 2
_NS = 16
_NW = _NC * _NS
_CHUNK = 128
_NBUF = 3


@functools.partial(jax.jit, static_argnames=("n_chunks", "d"))
def _sc_gather(table, idx3, n_chunks, d):
    n_rows = _NW * n_chunks * _CHUNK
    assert n_chunks % 2 == 0
    pairs = n_chunks // 2
    nbuf = min(_NBUF, pairs)

    @functools.partial(
        pl.kernel,
        out_type=jax.ShapeDtypeStruct((n_rows, d), jnp.float32),
        mesh=plsc.VectorSubcoreMesh(core_axis_name="c", subcore_axis_name="s"),
        scratch_types=[
            pltpu.VMEM((n_chunks, _CHUNK), jnp.int32),
            pltpu.VMEM((nbuf, 2 * _CHUNK, d), jnp.float32),
        ]
        + [pltpu.SemaphoreType.DMA] * (2 * nbuf),
    )
    def k(table_hbm, idx_hbm, out_hbm, idx_v, rows_v, *sems):
        sem_g, sem_s = sems[:nbuf], sems[nbuf:]
        wid = lax.axis_index("s") * _NC + lax.axis_index("c")
        pltpu.sync_copy(idx_hbm.at[wid], idx_v)
        base = wid * (n_chunks * _CHUNK)

        def gather(b, t, h):
            return pltpu.make_async_copy(
                table_hbm.at[idx_v.at[2 * t + h]],
                rows_v.at[b, pl.ds(h * _CHUNK, _CHUNK)], sem_g[b])

        def store(b, t):
            return pltpu.make_async_copy(
                rows_v.at[b],
                out_hbm.at[pl.ds(base + 2 * t * _CHUNK, 2 * _CHUNK)],
                sem_s[b])

        for b in range(nbuf):
            gather(b, b, 0).start()
            gather(b, b, 1).start()
        waited = 0
        for t in range(pairs):
            b = t % nbuf
            gather(b, t, 0).wait()
            gather(b, t, 1).wait()
            store(b, t).start()
            p = t - 1
            if p >= 0 and p + nbuf < pairs:
                store(p % nbuf, p).wait()
                gather(p % nbuf, p + nbuf, 0).start()
                gather(p % nbuf, p + nbuf, 1).start()
                waited = p + 1
        for t in range(waited, pairs):
            store(t % nbuf, t).wait()

    return k(table, idx3)


def kernel(indices, embedding_table):
    b, f = indices.shape
    v, d = embedding_table.shape
    n = b * f
    assert n % (_NW * _CHUNK) == 0
    n_chunks = n // (_NW * _CHUNK)
    idx3 = indices.T.reshape(_NW, n_chunks, _CHUNK).astype(jnp.int32)
    out = _sc_gather(embedding_table, idx3, n_chunks, d)
    return out.reshape(f, b, d).transpose(1, 0, 2)


# speedup vs baseline: 1.0378x; 1.0378x over previous
import functools

import jax
import jax.numpy as jnp
from jax import lax
from jax.experimental import pallas as pl
from jax.experimental.pallas import tpu as pltpu
from jax.experimental.pallas import tpu_sc as plsc

_NC = 2
_NS = 16
_NW = _NC * _NS
_CHUNK = 128
_NBUF = 7


@functools.partial(jax.jit, static_argnames=("n_chunks", "d"))
def _sc_gather(table, idx3, n_chunks, d):
    n_rows = _NW * n_chunks * _CHUNK
    nbuf = min(_NBUF, n_chunks)

    @functools.partial(
        pl.kernel,
        out_type=jax.ShapeDtypeStruct((n_rows, d), jnp.float32),
        mesh=plsc.VectorSubcoreMesh(core_axis_name="c", subcore_axis_name="s"),
        scratch_types=[
            pltpu.VMEM((n_chunks, _CHUNK), jnp.int32),
            pltpu.VMEM((nbuf, _CHUNK, d), jnp.float32),
        ]
        + [pltpu.SemaphoreType.DMA] * (2 * nbuf),
    )
    def k(table_hbm, idx_hbm, out_hbm, idx_v, rows_v, *sems):
        sem_g, sem_s = sems[:nbuf], sems[nbuf:]
        wid = lax.axis_index("s") * _NC + lax.axis_index("c")
        pltpu.sync_copy(idx_hbm.at[wid], idx_v)
        base = wid * (n_chunks * _CHUNK)

        def gather(b, j):
            return pltpu.make_async_copy(
                table_hbm.at[idx_v.at[j]], rows_v.at[b], sem_g[b])

        def store(b, j):
            return pltpu.make_async_copy(
                rows_v.at[b], out_hbm.at[pl.ds(base + j * _CHUNK, _CHUNK)],
                sem_s[b])

        for b in range(nbuf):
            gather(b, b).start()
        waited = 0
        for j in range(n_chunks):
            b = j % nbuf
            gather(b, j).wait()
            store(b, j).start()
            p = j - 1
            if p >= 0 and p + nbuf < n_chunks:
                store(p % nbuf, p).wait()
                gather(p % nbuf, p + nbuf).start()
                waited = p + 1
        for j in range(waited, n_chunks):
            store(j % nbuf, j).wait()

    return k(table, idx3)


def kernel(indices, embedding_table):
    b, f = indices.shape
    v, d = embedding_table.shape
    n = b * f
    assert n % (_NW * _CHUNK) == 0
    n_chunks = n // (_NW * _CHUNK)
    idx3 = indices.T.reshape(_NW, n_chunks, _CHUNK).astype(jnp.int32)
    out = _sc_gather(embedding_table, idx3, n_chunks, d)
    return out.reshape(f, b, d).transpose(1, 0, 2)


# speedup vs baseline: 1.0383x; 1.0004x over previous
import functools

import jax
import jax.numpy as jnp
from jax import lax
from jax.experimental import pallas as pl
from jax.experimental.pallas import tpu as pltpu
from jax.experimental.pallas import tpu_sc as plsc

_NC = 2
_NS = 16
_NW = _NC * _NS
_CHUNK = 128
_NBUF = 7


@functools.partial(jax.jit, static_argnames=("n_chunks", "d"))
def _sc_gather(table, idx3, n_chunks, d):
    n_rows = _NW * n_chunks * _CHUNK
    nbuf = min(_NBUF, n_chunks)

    @functools.partial(
        pl.kernel,
        out_type=jax.ShapeDtypeStruct((n_rows, d), jnp.float32),
        mesh=plsc.VectorSubcoreMesh(core_axis_name="c", subcore_axis_name="s"),
        scratch_types=[
            pltpu.VMEM((n_chunks, _CHUNK), jnp.int32),
            pltpu.VMEM((nbuf, _CHUNK, d), jnp.float32),
        ]
        + [pltpu.SemaphoreType.DMA] * (2 * nbuf),
    )
    def k(table_hbm, idx_hbm, out_hbm, idx_v, rows_v, *sems):
        sem_g, sem_s = sems[:nbuf], sems[nbuf:]
        wid = lax.axis_index("s") * _NC + lax.axis_index("c")
        pltpu.sync_copy(idx_hbm.at[wid], idx_v)
        base = wid * (n_chunks * _CHUNK)

        def gather(b, j):
            return pltpu.make_async_copy(
                table_hbm.at[idx_v.at[j]], rows_v.at[b], sem_g[b])

        def store(b, j):
            return pltpu.make_async_copy(
                rows_v.at[b], out_hbm.at[pl.ds(base + j * _CHUNK, _CHUNK)],
                sem_s[b])

        for b in range(nbuf):
            gather(b, b).start()
        waited = 0
        for j in range(n_chunks):
            b = j % nbuf
            gather(b, j).wait()
            store(b, j).start()
            p = j - 2
            if p >= 0 and p + nbuf < n_chunks:
                store(p % nbuf, p).wait()
                gather(p % nbuf, p + nbuf).start()
                waited = p + 1
        for j in range(waited, n_chunks):
            store(j % nbuf, j).wait()

    return k(table, idx3)


def kernel(indices, embedding_table):
    b, f = indices.shape
    v, d = embedding_table.shape
    n = b * f
    assert n % (_NW * _CHUNK) == 0
    n_chunks = n // (_NW * _CHUNK)
    idx3 = indices.T.reshape(_NW, n_chunks, _CHUNK).astype(jnp.int32)
    out = _sc_gather(embedding_table, idx3, n_chunks, d)
    return out.reshape(f, b, d).transpose(1, 0, 2)
